# Initial kernel scaffold; baseline (speedup 1.0000x reference)
import jax
import jax.numpy as jnp
from jax.experimental import pallas as pl


def kernel(x_lex, edge_index, batch, emb_table, W_gg, W_ih, W_hh, b_ih, b_hh, W1, b1, W2, b2):
    raise NotImplementedError("write your pallas kernel here")


# speedup vs baseline: 4.8425x; 4.8425x over previous
import functools

import jax
import jax.numpy as jnp
from jax import lax
from jax.experimental import pallas as pl
from jax.experimental.pallas import tpu as pltpu
from jax.experimental.pallas import tpu_sc as plsc

N = 10000
E = 320000
D = 128
G = 64
NC = 2
NS = 16
NW = NC * NS
LANES = 16

NPAD = 10240
RPT = NPAD // NW
EK = 80

EPT = E // NW
K = 80
NCH = EPT // K

PR = 313
PBUF = 320

_mesh = plsc.VectorSubcoreMesh(
    core_axis_name="c", subcore_axis_name="s", num_cores=NC, num_subcores=NS)


@functools.partial(
    pl.kernel,
    out_type=jax.ShapeDtypeStruct((NPAD, D), jnp.float32),
    mesh=_mesh,
    scratch_types=[
        pltpu.VMEM((EK,), jnp.int32),
        pltpu.VMEM((EK, D), jnp.float32),
        pltpu.SemaphoreType.DMA,
    ],
)
def _emb_gather(tab_hbm, idx_hbm, out_hbm, idx_v, rows_v, sem):
    c = lax.axis_index("c")
    s = lax.axis_index("s")
    wid = c * NS + s
    base = wid * RPT
    for j in range(RPT // EK):
        off = base + j * EK
        pltpu.sync_copy(idx_hbm.at[pl.ds(off, EK)], idx_v)
        pltpu.async_copy(tab_hbm.at[idx_v], rows_v, sem).wait()
        pltpu.sync_copy(rows_v, out_hbm.at[pl.ds(off, EK)])


@functools.partial(
    pl.kernel,
    out_type=jax.ShapeDtypeStruct((NC * N, D), jnp.float32),
    mesh=_mesh,
    scratch_types=[
        pltpu.VMEM_SHARED((N, D), jnp.float32),
        pltpu.VMEM((80, D), jnp.float32),
        pltpu.VMEM((K,), jnp.int32),
        pltpu.VMEM((K,), jnp.int32),
        pltpu.VMEM((K, D), jnp.float32),
        pltpu.SemaphoreType.DMA,
    ],
)
def _edge_aggr(m_hbm, src_hbm, dst_hbm, out_hbm, aggr_sh, zbuf, src_v, dst_v,
               rows_v, sem):
    c = lax.axis_index("c")
    s = lax.axis_index("s")
    wid = c * NS + s

    zv = jnp.zeros((LANES,), jnp.float32)
    for j in range(D // LANES):
        def zrow(i, _, j=j):
            zbuf[i, pl.ds(j * LANES, LANES)] = zv
            return 0
        lax.fori_loop(0, 80, zrow, 0)
    nck = jnp.where(s < 13, 8, 7)

    def zcopy(k, _):
        r = (s + NS * k) * 80
        pltpu.sync_copy(zbuf, aggr_sh.at[pl.ds(r, 80)])
        return 0

    lax.fori_loop(0, nck, zcopy, 0)
    plsc.subcore_barrier()

    ebase = wid * EPT

    def chunk(i, _):
        off = ebase + i * K
        pltpu.sync_copy(src_hbm.at[pl.ds(off, K)], src_v)
        pltpu.sync_copy(dst_hbm.at[pl.ds(off, K)], dst_v)
        pltpu.async_copy(m_hbm.at[src_v], rows_v, sem).wait()
        pltpu.sync_copy(rows_v, aggr_sh.at[dst_v], add=True)
        return 0

    lax.fori_loop(0, NCH, chunk, 0)
    plsc.subcore_barrier()

    def ocopy(k, _):
        r = (s + NS * k) * 80
        pltpu.sync_copy(aggr_sh.at[pl.ds(r, 80)],
                        out_hbm.at[pl.ds(c * N + r, 80)])
        return 0

    lax.fori_loop(0, nck, ocopy, 0)


@functools.partial(
    pl.kernel,
    out_type=jax.ShapeDtypeStruct((NW * G, D), jnp.float32),
    mesh=_mesh,
    scratch_types=[
        pltpu.VMEM((PBUF, D), jnp.float32),
        pltpu.VMEM((PBUF + LANES,), jnp.int32),
        pltpu.VMEM((G, D), jnp.float32),
    ],
)
def _seg_max(h_hbm, batch_hbm, out_hbm, h_v, b_v, acc):
    c = lax.axis_index("c")
    s = lax.axis_index("s")
    wid = c * NS + s

    ninf = jnp.full((LANES,), -jnp.inf, jnp.float32)
    for j in range(D // LANES):
        def irow(i, _, j=j):
            acc[i, pl.ds(j * LANES, LANES)] = ninf
            return 0
        lax.fori_loop(0, G, irow, 0)

    start = wid * PR
    nrows = jnp.minimum(PR, N - start)
    lstart = jnp.minimum((start // 8) * 8, N - PBUF)
    offs = start - lstart

    pltpu.sync_copy(h_hbm.at[pl.ds(lstart, PBUF)], h_v)
    pltpu.sync_copy(batch_hbm.at[pl.ds(lstart, PBUF)], b_v.at[pl.ds(0, PBUF)])

    def row(i, _):
        gid = b_v[pl.ds(offs + i, LANES)][0]
        for j in range(D // LANES):
            v = h_v[offs + i, pl.ds(j * LANES, LANES)]
            o = acc[gid, pl.ds(j * LANES, LANES)]
            acc[gid, pl.ds(j * LANES, LANES)] = jnp.maximum(o, v)
        return 0

    lax.fori_loop(0, nrows, row, 0)
    pltpu.sync_copy(acc, out_hbm.at[pl.ds(wid * G, G)])


def _mm_body(x_ref, w_ref, o_ref):
    o_ref[...] = jnp.dot(x_ref[...], w_ref[...],
                         preferred_element_type=jnp.float32)


_RB = 1000


def _msg_matmul(x, wg):
    return pl.pallas_call(
        _mm_body,
        grid=(N // _RB,),
        in_specs=[
            pl.BlockSpec((_RB, D), lambda i: (i, 0)),
            pl.BlockSpec((D, D), lambda i: (0, 0)),
        ],
        out_specs=pl.BlockSpec((_RB, D), lambda i: (i, 0)),
        out_shape=jax.ShapeDtypeStruct((N, D), jnp.float32),
    )(x, wg)


def _gru_body(a0_ref, a1_ref, h_ref, wih_ref, whh_ref, bih_ref, bhh_ref,
              wg_ref, h_out, m_out):
    a = a0_ref[...] + a1_ref[...]
    h = h_ref[...]
    gi = lax.dot_general(a, wih_ref[...], (((1,), (1,)), ((), ())),
                         preferred_element_type=jnp.float32) + bih_ref[...]
    gh = lax.dot_general(h, whh_ref[...], (((1,), (1,)), ((), ())),
                         preferred_element_type=jnp.float32) + bhh_ref[...]
    r = jax.nn.sigmoid(gi[:, 0:D] + gh[:, 0:D])
    z = jax.nn.sigmoid(gi[:, D:2 * D] + gh[:, D:2 * D])
    n = jnp.tanh(gi[:, 2 * D:3 * D] + r * gh[:, 2 * D:3 * D])
    hn = (1.0 - z) * n + z * h
    h_out[...] = hn
    m_out[...] = jnp.dot(hn, wg_ref[...], preferred_element_type=jnp.float32)


def _gru_step(aggr2, h, wih, whh, bih, bhh, wg_next):
    return pl.pallas_call(
        _gru_body,
        grid=(N // _RB,),
        in_specs=[
            pl.BlockSpec((_RB, D), lambda i: (i, 0)),
            pl.BlockSpec((_RB, D), lambda i: (i + N // _RB, 0)),
            pl.BlockSpec((_RB, D), lambda i: (i, 0)),
            pl.BlockSpec((3 * D, D), lambda i: (0, 0)),
            pl.BlockSpec((3 * D, D), lambda i: (0, 0)),
            pl.BlockSpec((1, 3 * D), lambda i: (0, 0)),
            pl.BlockSpec((1, 3 * D), lambda i: (0, 0)),
            pl.BlockSpec((D, D), lambda i: (0, 0)),
        ],
        out_specs=[
            pl.BlockSpec((_RB, D), lambda i: (i, 0)),
            pl.BlockSpec((_RB, D), lambda i: (i, 0)),
        ],
        out_shape=[
            jax.ShapeDtypeStruct((N, D), jnp.float32),
            jax.ShapeDtypeStruct((N, D), jnp.float32),
        ],
    )(aggr2, aggr2, h, wih, whh, bih, bhh, wg_next)


def _head_body(pp_ref, w1_ref, b1_ref, w2_ref, b2_ref, lg_out, hp_out):
    hp = jnp.max(pp_ref[...].reshape(NW, G, D), axis=0)
    hid = jax.nn.relu(
        lax.dot_general(hp, w1_ref[...], (((1,), (1,)), ((), ())),
                        preferred_element_type=jnp.float32) + b1_ref[...])
    lg = jnp.sum(hid * w2_ref[...], axis=1, keepdims=True) + b2_ref[...]
    lg_out[...] = lg
    hp_out[...] = hp


def _head(pool, w1, b1, w2, b2):
    return pl.pallas_call(
        _head_body,
        out_shape=[
            jax.ShapeDtypeStruct((G, 1), jnp.float32),
            jax.ShapeDtypeStruct((G, D), jnp.float32),
        ],
    )(pool, w1, b1, w2, b2)


def kernel(x_lex, edge_index, batch, emb_table, W_gg, W_ih, W_hh, b_ih, b_hh,
           W1, b1, W2, b2):
    src = edge_index[0]
    dst = edge_index[1]
    xl = jnp.concatenate(
        [x_lex.astype(jnp.int32), jnp.zeros((NPAD - N,), jnp.int32)])
    x = _emb_gather(emb_table, xl)[:N]
    h = x
    m = _msg_matmul(x, W_gg[0])
    bih = b_ih.reshape(1, 3 * D)
    bhh = b_hh.reshape(1, 3 * D)
    for i in range(3):
        aggr2 = _edge_aggr(m, src, dst)
        wg_next = W_gg[i + 1] if i < 2 else W_gg[0]
        h, m = _gru_step(aggr2, h, W_ih, W_hh, bih, bhh, wg_next)
    pool = _seg_max(h, batch)
    logits, h_pool = _head(pool, W1, b1.reshape(1, D // 4), W2,
                           b2.reshape(1, 1))
    return (logits, h_pool)
